# baseline (speedup 1.0000x reference)
import jax
import jax.numpy as jnp
from jax import lax
from jax.experimental import pallas as pl
from jax.experimental.pallas import tpu as pltpu

B = 256
C = 256
H = 25
W = 25
CROP = 7
ADD_U = 12
GB = 16
NS = B // GB


def _tc_body(s_ref, x_ref, mf_ref, o_ref, sc_ref):
    s = pl.program_id(0)

    @pl.when(s == 0)
    def _():
        sc_ref[...] = jnp.zeros((10, 10, GB, C), jnp.float32)

    bidx = lax.broadcasted_iota(jnp.int32, (CROP, CROP, GB, C), 2)
    zeros7 = jnp.zeros((CROP, CROP, GB, C), jnp.float32)
    acc = zeros7
    for k in range(GB):
        b = s * GB + k
        yc0 = s_ref[0, b]
        xc0 = s_ref[1, b]
        ro = s_ref[2, b]
        co = s_ref[3, b]
        xw = x_ref[pl.ds(yc0, CROP), pl.ds(xc0, CROP), :, :]
        sc_ref[pl.ds(ro, CROP), pl.ds(co, CROP), :, :] = xw
        win = sc_ref[3:3 + CROP, 0:CROP, :, :]
        acc = jnp.where(bidx == k, win, acc)
        sc_ref[pl.ds(ro, CROP), pl.ds(co, CROP), :, :] = zeros7
    o_ref[...] = acc * mf_ref[...]


def kernel(input_tensor, center_positions, mask, crop_mask):
    r = center_positions[:, 0].astype(jnp.int32)
    q = center_positions[:, 1].astype(jnp.int32)
    u3 = r - q // 2 + ADD_U - (CROP - 1) // 2
    vm3 = q - (CROP - 1) // 2
    yc0 = jnp.clip(u3, 0, H - CROP)
    xc0 = jnp.clip(vm3, 0, W - CROP)
    rowoff = 3 - (u3 - yc0)
    coloff = xc0 - vm3
    scals = jnp.stack([yc0, xc0, rowoff, coloff]).astype(jnp.int32)

    mask_factor = jnp.where(
        mask != 0, crop_mask, jnp.ones_like(crop_mask)).astype(jnp.float32)
    mf4 = jnp.broadcast_to(mask_factor[:, :, None, None], (CROP, CROP, GB, C))

    grid_spec = pltpu.PrefetchScalarGridSpec(
        num_scalar_prefetch=1,
        grid=(NS,),
        in_specs=[
            pl.BlockSpec((H, W, GB, C), lambda s, sc: (0, 0, s, 0)),
            pl.BlockSpec((CROP, CROP, GB, C), lambda s, sc: (0, 0, 0, 0)),
        ],
        out_specs=pl.BlockSpec((CROP, CROP, GB, C), lambda s, sc: (0, 0, s, 0)),
        scratch_shapes=[
            pltpu.VMEM((10, 10, GB, C), jnp.float32),
        ],
    )
    out_t = pl.pallas_call(
        _tc_body,
        grid_spec=grid_spec,
        out_shape=jax.ShapeDtypeStruct((CROP, CROP, B, C), jnp.float32),
    )(scals, input_tensor.transpose(2, 3, 0, 1), mf4)
    return (out_t.transpose(2, 3, 0, 1), crop_mask)


# speedup vs baseline: 1.0057x; 1.0057x over previous
import jax
import jax.numpy as jnp
from jax import lax
from jax.experimental import pallas as pl
from jax.experimental.pallas import tpu as pltpu

B = 256
C = 256
H = 25
W = 25
CROP = 7
ADD_U = 12
GB = 8
NS = B // GB


def _tc_body(s_ref, x_ref, mf_ref, o_ref, sc_ref):
    s = pl.program_id(0)

    @pl.when(s == 0)
    def _():
        sc_ref[...] = jnp.zeros((10, 10, GB, C), jnp.float32)

    bidx = lax.broadcasted_iota(jnp.int32, (CROP, CROP, GB, C), 2)
    zeros7 = jnp.zeros((CROP, CROP, GB, C), jnp.float32)
    acc = zeros7
    for k in range(GB):
        b = s * GB + k
        yc0 = s_ref[0, b]
        xc0 = s_ref[1, b]
        ro = s_ref[2, b]
        co = s_ref[3, b]
        xw = x_ref[pl.ds(yc0, CROP), pl.ds(xc0, CROP), :, :]
        sc_ref[pl.ds(ro, CROP), pl.ds(co, CROP), :, :] = xw
        win = sc_ref[3:3 + CROP, 0:CROP, :, :]
        acc = jnp.where(bidx == k, win, acc)
        sc_ref[pl.ds(ro, CROP), pl.ds(co, CROP), :, :] = zeros7
    o_ref[...] = acc * mf_ref[...]


def kernel(input_tensor, center_positions, mask, crop_mask):
    r = center_positions[:, 0].astype(jnp.int32)
    q = center_positions[:, 1].astype(jnp.int32)
    u3 = r - q // 2 + ADD_U - (CROP - 1) // 2
    vm3 = q - (CROP - 1) // 2
    yc0 = jnp.clip(u3, 0, H - CROP)
    xc0 = jnp.clip(vm3, 0, W - CROP)
    rowoff = 3 - (u3 - yc0)
    coloff = xc0 - vm3
    scals = jnp.stack([yc0, xc0, rowoff, coloff]).astype(jnp.int32)

    mask_factor = jnp.where(
        mask != 0, crop_mask, jnp.ones_like(crop_mask)).astype(jnp.float32)
    mf4 = jnp.broadcast_to(mask_factor[:, :, None, None], (CROP, CROP, GB, C))

    grid_spec = pltpu.PrefetchScalarGridSpec(
        num_scalar_prefetch=1,
        grid=(NS,),
        in_specs=[
            pl.BlockSpec((H, W, GB, C), lambda s, sc: (0, 0, s, 0)),
            pl.BlockSpec((CROP, CROP, GB, C), lambda s, sc: (0, 0, 0, 0)),
        ],
        out_specs=pl.BlockSpec((CROP, CROP, GB, C), lambda s, sc: (0, 0, s, 0)),
        scratch_shapes=[
            pltpu.VMEM((10, 10, GB, C), jnp.float32),
        ],
    )
    out_t = pl.pallas_call(
        _tc_body,
        grid_spec=grid_spec,
        out_shape=jax.ShapeDtypeStruct((CROP, CROP, B, C), jnp.float32),
    )(scals, input_tensor.transpose(2, 3, 0, 1), mf4)
    return (out_t.transpose(2, 3, 0, 1), crop_mask)
